# Initial kernel scaffold; baseline (speedup 1.0000x reference)
import jax
import jax.numpy as jnp
from jax.experimental import pallas as pl


def kernel(distances, sbf, species, idx_i, idx_j, reduce_to_ji, expand_to_kj, rbf_freq, embed_vect, emb_rbf_W, emb_concat_W, emb_concat_b, int_rbf1_W, int_rbf2_W, int_sbf1_W, int_sbf2_W, int_dense_kj_W, int_dense_kj_b, int_down_W, int_up_W, int_dense_ji_W, int_dense_ji_b, int_res_before_W, int_res_before_b, int_final_W, int_final_b, int_res_after_W, int_res_after_b, out_rbf_W, out_up_W, out_dense_W, out_dense_b, out_final_W):
    raise NotImplementedError("write your pallas kernel here")


# speedup vs baseline: 4.1559x; 4.1559x over previous
import functools

import jax
import jax.numpy as jnp
from jax import lax
from jax.experimental import pallas as pl
from jax.experimental.pallas import tpu as pltpu
from jax.experimental.pallas import tpu_sc as plsc

N = 10000
E = 320000
T = 640000
CUTOFF = 5.0

NC = 2
NS = 16
NW = NC * NS

_MESH = dict(core_axis_name="c", subcore_axis_name="s", num_cores=NC,
             num_subcores=NS)


def _swish(x):
    return x / (1.0 + jnp.exp(-x))


_EW = E // NW


def _sc_species_body(species_hbm, idx_i_hbm, idx_j_hbm, si_hbm, sj_hbm,
                     idx_v, out_v):
    c = lax.axis_index("c")
    s = lax.axis_index("s")
    wid = s * NC + c
    base = wid * _EW

    def gather_one(ih, oh):
        pltpu.sync_copy(ih.at[pl.ds(base, _EW)], idx_v)
        pltpu.sync_copy(species_hbm.at[idx_v], out_v)
        pltpu.sync_copy(out_v, oh.at[pl.ds(base, _EW)])

    gather_one(idx_i_hbm, si_hbm)
    gather_one(idx_j_hbm, sj_hbm)


def _sc_species(species, idx_i, idx_j):
    f = pl.kernel(
        _sc_species_body,
        out_type=[jax.ShapeDtypeStruct((E,), jnp.int32),
                  jax.ShapeDtypeStruct((E,), jnp.int32)],
        mesh=plsc.VectorSubcoreMesh(**_MESH),
        scratch_types=[pltpu.VMEM((_EW,), jnp.int32),
                       pltpu.VMEM((_EW,), jnp.int32)],
    )
    return f(species, idx_i, idx_j)


_NB = 400
_SB = 80
_NPAD = 10240
_NROW = _NPAD // NS


def _sc_segn_body(tv_hbm, idx_hbm, out_hbm, acc_sh, vbuf, ibuf):
    c = lax.axis_index("c")
    s = lax.axis_index("s")
    wid = s * NC + c
    base = wid * _EW

    def zbody(i, carry):
        z = jnp.zeros((16,), jnp.float32)
        vbuf[i, pl.ds(0, 16)] = z
        vbuf[i, pl.ds(16, 16)] = z
        vbuf[i, pl.ds(32, 16)] = z
        vbuf[i, pl.ds(48, 16)] = z
        return carry

    lax.fori_loop(0, _SB, zbody, 0, unroll=4)

    def zcopy(j, carry):
        pltpu.sync_copy(vbuf, acc_sh.at[pl.ds(s * _NROW + j * _SB, _SB)])
        return carry

    lax.fori_loop(0, _NROW // _SB, zcopy, 0)
    plsc.subcore_barrier()

    def batch(k, carry):
        off = base + k * _SB
        pltpu.sync_copy(tv_hbm.at[pl.ds(off, _SB)], vbuf)
        pltpu.sync_copy(idx_hbm.at[pl.ds(off, _SB)], ibuf)
        pltpu.sync_copy(vbuf, acc_sh.at[ibuf], add=True)
        return carry

    lax.fori_loop(0, _EW // _SB, batch, 0)
    plsc.subcore_barrier()
    pltpu.sync_copy(acc_sh.at[pl.ds(s * _NROW, _NROW)],
                    out_hbm.at[c, pl.ds(s * _NROW, _NROW), :])


def _sc_segn(tvals, idx_flat):
    f = pl.kernel(
        _sc_segn_body,
        out_type=jax.ShapeDtypeStruct((NC, _NPAD, 64), jnp.float32),
        mesh=plsc.VectorSubcoreMesh(**_MESH),
        scratch_types=[pltpu.VMEM_SHARED((_NPAD, 64), jnp.float32),
                       pltpu.VMEM((_SB, 64), jnp.float32),
                       pltpu.VMEM((_SB,), jnp.int32)],
    )
    return f(tvals, idx_flat)


_TWG = T // NW


def _sc_gather_body(ma_hbm, exp_hbm, out_hbm, ibuf, gbuf):
    c = lax.axis_index("c")
    s = lax.axis_index("s")
    wid = s * NC + c
    base = wid * _TWG

    def batch(k, carry):
        off = base + k * _SB
        pltpu.sync_copy(exp_hbm.at[pl.ds(off, _SB)], ibuf)
        pltpu.sync_copy(ma_hbm.at[ibuf], gbuf)
        pltpu.sync_copy(gbuf, out_hbm.at[pl.ds(off, _SB), :])
        return carry

    lax.fori_loop(0, _TWG // _SB, batch, 0)


def _sc_gather(ma, exp):
    f = pl.kernel(
        _sc_gather_body,
        out_type=jax.ShapeDtypeStruct((T, 128), jnp.float32),
        mesh=plsc.VectorSubcoreMesh(**_MESH),
        scratch_types=[pltpu.VMEM((_SB,), jnp.int32),
                       pltpu.VMEM((_SB, 128), jnp.float32)],
    )
    return f(ma, exp)


_EC = 53632
_NCH = 6
_EPAD = _NCH * _EC
_ACC = _EC + 128
_ZR = 56
_TRASH = _EC
_TPT = T // NS


def _sc_segt_body(cv_hbm, red_hbm, agg_hbm, acc_sh, vbuf, ibuf, zbuf):
    c = lax.axis_index("c")
    s = lax.axis_index("s")

    def zb(i, carry):
        z = jnp.zeros((16,), jnp.float32)
        zbuf[i, pl.ds(0, 16)] = z
        zbuf[i, pl.ds(16, 16)] = z
        return carry

    lax.fori_loop(0, _ZR, zb, 0, unroll=4)

    def one_pass(p, carry):
        chunk = p * NC + c
        lo = chunk * _EC

        def zcopy(j, carry2):
            pltpu.sync_copy(
                zbuf, acc_sh.at[pl.ds(s * (_ACC // NS) + j * _ZR, _ZR)])
            return carry2

        lax.fori_loop(0, _ACC // NS // _ZR, zcopy, 0)
        plsc.subcore_barrier()

        def batch(k, carry2):
            off = s * _TPT + k * _SB
            pltpu.sync_copy(cv_hbm.at[pl.ds(off, _SB), :], vbuf)
            pltpu.sync_copy(red_hbm.at[pl.ds(off, _SB)], ibuf)
            for jj in range(_SB // 16):
                rv = ibuf[pl.ds(jj * 16, 16)]
                inb = (rv >= lo) & (rv < lo + _EC)
                ibuf[pl.ds(jj * 16, 16)] = jnp.where(inb, rv - lo, _TRASH)
            pltpu.sync_copy(vbuf, acc_sh.at[ibuf], add=True)
            return carry2

        lax.fori_loop(0, _TPT // _SB, batch, 0)
        plsc.subcore_barrier()

        rows = _EC // NS
        pltpu.sync_copy(acc_sh.at[pl.ds(s * rows, rows)],
                        agg_hbm.at[pl.ds(lo + s * rows, rows), :])
        plsc.subcore_barrier()
        return carry

    lax.fori_loop(0, _NCH // NC, one_pass, 0)


def _sc_segt(contrib, red_flat):
    f = pl.kernel(
        _sc_segt_body,
        out_type=jax.ShapeDtypeStruct((_EPAD, 32), jnp.float32),
        mesh=plsc.VectorSubcoreMesh(**_MESH),
        scratch_types=[pltpu.VMEM_SHARED((_ACC, 32), jnp.float32),
                       pltpu.VMEM((_SB, 32), jnp.float32),
                       pltpu.VMEM((_SB,), jnp.int32),
                       pltpu.VMEM((_ZR, 32), jnp.float32)],
    )
    return f(contrib, red_flat)


_EB = 3200
_TB = 6400


def _rbf_from_dist(d, freq):
    x = d * (1.0 / CUTOFF)
    p = 6.0
    a = -(p + 1.0) * (p + 2.0) / 2.0
    b = p * (p + 2.0)
    cc = -p * (p + 1.0) / 2.0
    x6 = x * x * x
    x6 = x6 * x6
    env = jnp.where(x < 1.0, 1.0 + x6 * (a + x * (b + cc * x)), 0.0)
    return env * jnp.sqrt(2.0 / CUTOFF) * jnp.sin(freq * x) / d


def _tc_edge0_body(d_ref, si_ref, sj_ref, t1_ref, t2_ref, w3_ref, bb_ref,
                   freq_ref, wo_ref, m_ref, rbf_ref, t0_ref):
    d = d_ref[0, 0].reshape(_EB, 1)
    rbf = _rbf_from_dist(d, freq_ref[:])
    ioto = lax.broadcasted_iota(jnp.int32, (_EB, 16), 1)
    ohi = (si_ref[0, 0].reshape(_EB, 1) == ioto).astype(jnp.float32)
    ohj = (sj_ref[0, 0].reshape(_EB, 1) == ioto).astype(jnp.float32)
    pre = (jnp.dot(ohi, t1_ref[:], preferred_element_type=jnp.float32)
           + jnp.dot(ohj, t2_ref[:], preferred_element_type=jnp.float32)
           + jnp.dot(rbf, w3_ref[:], preferred_element_type=jnp.float32)
           + bb_ref[:])
    m = _swish(pre)
    m_ref[:] = m
    rbf_ref[:] = rbf
    t0_ref[:] = m * jnp.dot(rbf, wo_ref[:], preferred_element_type=jnp.float32)


def _tc_edge0(distances, s_i, s_j, t1, t2, w3, bb, freq, wo):
    g = E // _EB
    distances = distances.reshape(g, 1, _EB)
    s_i = s_i.reshape(g, 1, _EB)
    s_j = s_j.reshape(g, 1, _EB)
    grid = (g,)
    vec_spec = pl.BlockSpec((1, 1, _EB), lambda i: (i, 0, 0))
    full = lambda a: pl.BlockSpec(a.shape, lambda i: (0,) * a.ndim)
    return pl.pallas_call(
        _tc_edge0_body,
        grid=grid,
        in_specs=[vec_spec, vec_spec, vec_spec, full(t1), full(t2), full(w3),
                  full(bb), full(freq), full(wo)],
        out_specs=[pl.BlockSpec((_EB, 64), lambda i: (i, 0)),
                   pl.BlockSpec((_EB, 16), lambda i: (i, 0)),
                   pl.BlockSpec((_EB, 64), lambda i: (i, 0))],
        out_shape=[jax.ShapeDtypeStruct((E, 64), jnp.float32),
                   jax.ShapeDtypeStruct((E, 16), jnp.float32),
                   jax.ShapeDtypeStruct((E, 64), jnp.float32)],
    )(distances, s_i, s_j, t1, t2, w3, bb, freq, wo)


def _tc_sbfmul_body(sbf_ref, gath_ref, w_ref, o_ref):
    o_ref[:] = gath_ref[:, :32] * jnp.dot(sbf_ref[:], w_ref[:],
                                          preferred_element_type=jnp.float32)


def _tc_sbfmul(sbf, gath, w):
    grid = (T // _TB,)
    full = lambda a: pl.BlockSpec(a.shape, lambda i: (0,) * a.ndim)
    return pl.pallas_call(
        _tc_sbfmul_body,
        grid=grid,
        in_specs=[pl.BlockSpec((_TB, 16), lambda i: (i, 0)),
                  pl.BlockSpec((_TB, 128), lambda i: (i, 0)), full(w)],
        out_specs=pl.BlockSpec((_TB, 32), lambda i: (i, 0)),
        out_shape=jax.ShapeDtypeStruct((T, 32), jnp.float32),
    )(sbf, gath, w)


def _tc_ang_body(m_ref, rbf_ref, wkj_ref, bkj_ref, wr_ref, wd_ref, o_ref):
    m = m_ref[:]
    ma = _swish(jnp.dot(m, wkj_ref[:], preferred_element_type=jnp.float32)
                + bkj_ref[:])
    ma = ma * jnp.dot(rbf_ref[:], wr_ref[:], preferred_element_type=jnp.float32)
    v = _swish(jnp.dot(ma, wd_ref[:], preferred_element_type=jnp.float32))
    o_ref[:] = jnp.concatenate(
        [v, jnp.zeros((v.shape[0], 96), jnp.float32)], axis=1)


def _tc_ang(m, rbf, wkj, bkj, wr, wd):
    grid = (E // _EB,)
    full = lambda a: pl.BlockSpec(a.shape, lambda i: (0,) * a.ndim)
    return pl.pallas_call(
        _tc_ang_body,
        grid=grid,
        in_specs=[pl.BlockSpec((_EB, 64), lambda i: (i, 0)),
                  pl.BlockSpec((_EB, 16), lambda i: (i, 0)),
                  full(wkj), full(bkj), full(wr), full(wd)],
        out_specs=pl.BlockSpec((_EB, 128), lambda i: (i, 0)),
        out_shape=jax.ShapeDtypeStruct((E, 128), jnp.float32),
    )(m, rbf, wkj, bkj, wr, wd)


def _tc_post_body(m_ref, rbf_ref, agg_ref, wup_ref, wji_ref, bji_ref,
                  wrb_ref, brb_ref, wfin_ref, bfin_ref, wra_ref, bra_ref,
                  wo_ref, mo_ref, t_ref):
    dot = lambda a, b: jnp.dot(a, b, preferred_element_type=jnp.float32)
    m = m_ref[:]
    prop = _swish(dot(agg_ref[:], wup_ref[:]))
    mc = _swish(dot(m, wji_ref[:]) + bji_ref[:]) + prop
    h = _swish(dot(mc, wrb_ref[0]) + brb_ref[0])
    h = _swish(dot(h, wrb_ref[1]) + brb_ref[1])
    mc = mc + h
    mc = _swish(dot(mc, wfin_ref[:]) + bfin_ref[:])
    mn = mc + m
    for r in range(2):
        h = _swish(dot(mn, wra_ref[r, 0]) + bra_ref[r, 0])
        h = _swish(dot(h, wra_ref[r, 1]) + bra_ref[r, 1])
        mn = mn + h
    mo_ref[:] = mn
    t_ref[:] = mn * dot(rbf_ref[:], wo_ref[:])


def _tc_post(m, rbf, agg, wup, wji, bji, wrb, brb, wfin, bfin, wra, bra, wo):
    grid = (E // _EB,)
    full = lambda a: pl.BlockSpec(a.shape, lambda i: (0,) * a.ndim)
    return pl.pallas_call(
        _tc_post_body,
        grid=grid,
        in_specs=[pl.BlockSpec((_EB, 64), lambda i: (i, 0)),
                  pl.BlockSpec((_EB, 16), lambda i: (i, 0)),
                  pl.BlockSpec((_EB, 32), lambda i: (i, 0)),
                  full(wup), full(wji), full(bji), full(wrb), full(brb),
                  full(wfin), full(bfin), full(wra), full(bra), full(wo)],
        out_specs=[pl.BlockSpec((_EB, 64), lambda i: (i, 0)),
                   pl.BlockSpec((_EB, 64), lambda i: (i, 0))],
        out_shape=[jax.ShapeDtypeStruct((E, 64), jnp.float32),
                   jax.ShapeDtypeStruct((E, 64), jnp.float32)],
    )(m, rbf, agg, wup, wji, bji, wrb, brb, wfin, bfin, wra, bra, wo)


def _tc_out_body(s0_ref, s1_ref, s2_ref, wup_ref, wd_ref, bd_ref, wf_ref,
                 o_ref):
    dot = lambda a, b: jnp.dot(a, b, preferred_element_type=jnp.float32)
    tot = jnp.zeros((1, 1), jnp.float32)
    for k, s_ref in enumerate((s0_ref, s1_ref, s2_ref)):
        sacc = (s_ref[0] + s_ref[1])[:N]
        u = dot(sacc, wup_ref[k])
        for j in range(2):
            u = _swish(dot(u, wd_ref[k, j]) + bd_ref[k, j])
        col = jnp.sum(u, axis=0, keepdims=True)
        tot = tot + dot(col, wf_ref[k])
    o_ref[:] = tot


def _tc_out(s0, s1, s2, wup, wd, bd, wf):
    full = lambda a: pl.BlockSpec(a.shape, lambda *_: (0,) * a.ndim)
    out = pl.pallas_call(
        _tc_out_body,
        in_specs=[full(s0), full(s1), full(s2), full(wup), full(wd), full(bd),
                  full(wf)],
        out_specs=pl.BlockSpec((1, 1), lambda *_: (0, 0)),
        out_shape=jax.ShapeDtypeStruct((1, 1), jnp.float32),
    )(s0, s1, s2, wup, wd, bd, wf)
    return out[0, 0]


def kernel(distances, sbf, species, idx_i, idx_j, reduce_to_ji, expand_to_kj,
           rbf_freq, embed_vect, emb_rbf_W, emb_concat_W, emb_concat_b,
           int_rbf1_W, int_rbf2_W, int_sbf1_W, int_sbf2_W, int_dense_kj_W,
           int_dense_kj_b, int_down_W, int_up_W, int_dense_ji_W,
           int_dense_ji_b, int_res_before_W, int_res_before_b, int_final_W,
           int_final_b, int_res_after_W, int_res_after_b, out_rbf_W,
           out_up_W, out_dense_W, out_dense_b, out_final_W):
    f32 = jnp.float32
    species = species.astype(jnp.int32)
    idx_i = idx_i.astype(jnp.int32)
    idx_j = idx_j.astype(jnp.int32)
    reduce_to_ji = reduce_to_ji.astype(jnp.int32)
    expand_to_kj = expand_to_kj.astype(jnp.int32)

    t1 = embed_vect @ emb_concat_W[:32]
    t2 = embed_vect @ emb_concat_W[32:64]
    w3 = emb_rbf_W @ emb_concat_W[64:]
    bb = emb_concat_b.reshape(1, 64)
    freq = rbf_freq.reshape(1, 16)
    w_rbf = jnp.einsum("bij,bjk->bik", int_rbf1_W, int_rbf2_W)
    w_sbf = jnp.einsum("bij,bjk->bik", int_sbf1_W, int_sbf2_W)

    s_i, s_j = _sc_species(species, idx_i, idx_j)
    m, rbf, t0 = _tc_edge0(distances, s_i, s_j, t1, t2, w3, bb, freq,
                           out_rbf_W[0])

    def _segn_fb(t):
        s = jax.ops.segment_sum(t, idx_i, num_segments=N)
        s = jnp.pad(s, ((0, _NPAD - N), (0, 0)))
        return jnp.stack([s, jnp.zeros_like(s)])

    s_parts = [_segn_fb(t0)]
    for blk in range(2):
        ma = _tc_ang(m, rbf, int_dense_kj_W[blk],
                     int_dense_kj_b[blk].reshape(1, 64), w_rbf[blk],
                     int_down_W[blk])
        gath = _sc_gather(ma, expand_to_kj)
        contrib = _tc_sbfmul(sbf, gath, w_sbf[blk])
        agg = jax.ops.segment_sum(contrib, reduce_to_ji, num_segments=E)
        m, t = _tc_post(m, rbf, agg, int_up_W[blk], int_dense_ji_W[blk],
                        int_dense_ji_b[blk].reshape(1, 64),
                        int_res_before_W[blk, 0], int_res_before_b[blk, 0],
                        int_final_W[blk], int_final_b[blk].reshape(1, 64),
                        int_res_after_W[blk], int_res_after_b[blk],
                        out_rbf_W[blk + 1])
        s_parts.append(_segn_fb(t))

    return _tc_out(s_parts[0], s_parts[1], s_parts[2], out_up_W, out_dense_W,
                   out_dense_b, out_final_W).astype(f32)
